# baseline (speedup 1.0000x reference)
import functools

import jax
import jax.numpy as jnp
from jax import lax
from jax.experimental import pallas as pl
from jax.experimental.pallas import tpu as pltpu
from jax.experimental.pallas import tpu_sc as plsc

HIDDEN = 64
LANES = 16
NCORES = 2
NSUBCORES = 16
NW = NCORES * NSUBCORES

SUB = 128
RING = 4

_GATHER_DNUMS = lax.GatherDimensionNumbers(
    offset_dims=(), collapsed_slice_dims=(0,), start_index_map=(0,)
)


def _perm(v, idx16):
    return lax.gather(
        v,
        idx16[:, None],
        _GATHER_DNUMS,
        slice_sizes=(1,),
        mode=lax.GatherScatterMode.PROMISE_IN_BOUNDS,
    )


def _rsqrt(s):
    i = lax.bitcast_convert_type(s, jnp.int32)
    i = jnp.int32(0x5F3759DF) - lax.shift_right_logical(i, 1)
    y = lax.bitcast_convert_type(i, jnp.float32)
    h = 0.5 * s
    for _ in range(2):
        y = y * (1.5 - h * y * y)
    return y


def _make_kernel(batch, seq):
    units = seq
    groups = units // RING
    out_rows = seq * HIDDEN
    mesh = plsc.VectorSubcoreMesh(core_axis_name="c", subcore_axis_name="s")

    @functools.partial(
        pl.kernel,
        mesh=mesh,
        out_type=jax.ShapeDtypeStruct((out_rows, batch), jnp.float32),
        scratch_types=[
            pltpu.VMEM((units, SUB), jnp.int32),
            pltpu.VMEM((RING, SUB), jnp.int32),
            pltpu.VMEM((RING, SUB, 128), jnp.float32),
            pltpu.VMEM((RING, HIDDEN, SUB), jnp.float32),
            [pltpu.SemaphoreType.DMA] * RING,
            [pltpu.SemaphoreType.DMA] * RING,
        ],
        compiler_params=pltpu.CompilerParams(
            use_tc_tiling_on_sc=True, needs_layout_passes=False
        ),
    )
    def gather_norm(idx_hbm, table_hbm, out_hbm, idx_v, half_v, staged, sbuf,
                    gsems, ssems):
        wid = lax.axis_index("s") * NCORES + lax.axis_index("c")
        bcol = wid * SUB
        lane = lax.iota(jnp.int32, LANES)
        perms = [lane ^ (1 << k) for k in (3, 2, 1, 0)]
        rowidx = [lane + k * LANES for k in range(4)]

        pltpu.sync_copy(idx_hbm.at[pl.ds(wid * units, units)], idx_v)

        def prep_and_fire(u, b):
            for k in range(SUB // LANES):
                half_v[b, pl.ds(k * LANES, LANES)] = lax.shift_right_logical(
                    idx_v[u, pl.ds(k * LANES, LANES)], 1
                )
            pltpu.async_copy(table_hbm.at[half_v.at[b]], staged.at[b], gsems[b])

        for b in range(RING):
            prep_and_fire(b, b)

        def group_body(grp, carry):
            for b in range(RING):
                u = grp * RING + b
                pltpu.make_async_copy(
                    table_hbm.at[half_v.at[b]], staged.at[b], gsems[b]
                ).wait()

                @pl.when(grp > 0)
                def _():
                    pltpu.make_async_copy(
                        sbuf.at[b],
                        out_hbm.at[pl.ds(0, HIDDEN), pl.ds(bcol, SUB)],
                        ssems[b],
                    ).wait()

                bvec = jnp.full((LANES,), b, jnp.int32)

                def row_block(i16, _):
                    base = i16 * LANES
                    cbv = lax.shift_left(idx_v[u, pl.ds(base, LANES)] & 1, 6)
                    for r in range(LANES):
                        j = base + r
                        cb = cbv[r]
                        v = [
                            staged[b, j, pl.ds(cb + k * LANES, LANES)]
                            for k in range(4)
                        ]
                        q = v[0] * v[0] + v[1] * v[1] + v[2] * v[2] + v[3] * v[3]
                        for p in perms:
                            q = q + _perm(q, p)
                        y = _rsqrt(q)
                        jvec = jnp.full((LANES,), j, jnp.int32)
                        for k in range(4):
                            plsc.store_scatter(
                                sbuf, [bvec, rowidx[k], jvec], v[k] * y
                            )
                    return 0

                lax.fori_loop(0, SUB // LANES, row_block, 0)

                pltpu.async_copy(
                    sbuf.at[b],
                    out_hbm.at[pl.ds(u * HIDDEN, HIDDEN), pl.ds(bcol, SUB)],
                    ssems[b],
                )

                @pl.when(grp < groups - 1)
                def _():
                    prep_and_fire(u + RING, b)
            return carry

        lax.fori_loop(0, groups, group_body, 0)

        for b in range(RING):
            pltpu.make_async_copy(
                sbuf.at[b],
                out_hbm.at[pl.ds(0, HIDDEN), pl.ds(bcol, SUB)],
                ssems[b],
            ).wait()

    return gather_norm


def kernel(inputs, table):
    batch, seq = inputs.shape
    idx_t = (
        inputs.T.reshape(seq, batch // SUB, SUB)
        .swapaxes(0, 1)
        .reshape(seq * batch // SUB, SUB)
    )
    table2 = table.reshape(table.shape[0] // 2, 128)
    out = _make_kernel(batch, seq)(idx_t, table2)
    return out.reshape(seq, HIDDEN, batch).transpose(2, 0, 1)


# speedup vs baseline: 1.2217x; 1.2217x over previous
import functools

import jax
import jax.numpy as jnp
from jax import lax
from jax.experimental import pallas as pl
from jax.experimental.pallas import tpu as pltpu
from jax.experimental.pallas import tpu_sc as plsc

HIDDEN = 64
LANES = 16
NCORES = 2
NSUBCORES = 16
NW = NCORES * NSUBCORES

SUB = 128
RING = 4
SPAD = SUB + 1

_GATHER_DNUMS = lax.GatherDimensionNumbers(
    offset_dims=(), collapsed_slice_dims=(0,), start_index_map=(0,)
)


def _perm(v, idx16):
    return lax.gather(
        v,
        idx16[:, None],
        _GATHER_DNUMS,
        slice_sizes=(1,),
        mode=lax.GatherScatterMode.PROMISE_IN_BOUNDS,
    )


def _rsqrt(s):
    i = lax.bitcast_convert_type(s, jnp.int32)
    i = jnp.int32(0x5F3759DF) - lax.shift_right_logical(i, 1)
    y = lax.bitcast_convert_type(i, jnp.float32)
    h = 0.5 * s
    for _ in range(2):
        y = y * (1.5 - h * y * y)
    return y


def _make_kernel(batch, seq):
    units = seq
    groups = units // RING
    out_rows = seq * HIDDEN
    mesh = plsc.VectorSubcoreMesh(core_axis_name="c", subcore_axis_name="s")

    @functools.partial(
        pl.kernel,
        mesh=mesh,
        out_type=jax.ShapeDtypeStruct((out_rows, batch), jnp.float32),
        scratch_types=[
            pltpu.VMEM((RING, 1, SUB), jnp.int32),
            pltpu.VMEM((RING, SUB), jnp.int32),
            pltpu.VMEM((RING, SUB), jnp.int32),
            pltpu.VMEM((RING, SUB, 128), jnp.float32),
            pltpu.VMEM((2, HIDDEN, SPAD), jnp.float32),
            [pltpu.SemaphoreType.DMA] * RING,
            [pltpu.SemaphoreType.DMA] * RING,
            [pltpu.SemaphoreType.DMA] * 2,
        ],
        compiler_params=pltpu.CompilerParams(
            use_tc_tiling_on_sc=True, needs_layout_passes=False
        ),
    )
    def gather_norm(idx_hbm, table_hbm, out_hbm, idx_v, half_v, lsb_v, staged,
                    sbuf, isems, gsems, ssems):
        wid = lax.axis_index("s") * NCORES + lax.axis_index("c")
        bcol = wid * SUB
        irow = wid * units
        lane = lax.iota(jnp.int32, LANES)
        perms = [lane ^ (1 << k) for k in (3, 2, 1, 0)]
        rowidx = [lane + k * LANES for k in range(4)]

        def fire_idx(u, b):
            pltpu.async_copy(
                idx_hbm.at[pl.ds(irow + u, 1)], idx_v.at[b], isems[b]
            )

        def halve_and_fire(b):
            for k in range(SUB // LANES):
                iv = idx_v[b, 0, pl.ds(k * LANES, LANES)]
                half_v[b, pl.ds(k * LANES, LANES)] = lax.shift_right_logical(
                    iv, 1
                )
                lsb_v[b, pl.ds(k * LANES, LANES)] = lax.shift_left(iv & 1, 6)
            pltpu.async_copy(table_hbm.at[half_v.at[b]], staged.at[b], gsems[b])

        for b in range(RING):
            fire_idx(b, b)
        for b in range(RING):
            pltpu.make_async_copy(
                idx_hbm.at[pl.ds(irow, 1)], idx_v.at[b], isems[b]
            ).wait()
            halve_and_fire(b)
            fire_idx(b + RING, b)

        def group_body(grp, carry):
            for b in range(RING):
                u = grp * RING + b
                sb = b % 2
                pltpu.make_async_copy(
                    table_hbm.at[half_v.at[b]], staged.at[b], gsems[b]
                ).wait()

                def wait_store():
                    pltpu.make_async_copy(
                        sbuf.at[sb, :, pl.ds(0, SUB)],
                        out_hbm.at[pl.ds(0, HIDDEN), pl.ds(bcol, SUB)],
                        ssems[sb],
                    ).wait()

                if b >= 2:
                    wait_store()
                else:
                    @pl.when(grp > 0)
                    def _():
                        wait_store()

                @plsc.parallel_loop(0, SUB // LANES, step=1, unroll=2)
                def row_block(i16):
                    base = i16 * LANES
                    cbv = lsb_v[b, pl.ds(base, LANES)]
                    for r in range(LANES):
                        j = base + r
                        cb = cbv[r]
                        v = [
                            staged[b, j, pl.ds(cb + k * LANES, LANES)]
                            for k in range(4)
                        ]
                        q = v[0] * v[0] + v[1] * v[1] + v[2] * v[2] + v[3] * v[3]
                        for p in perms:
                            q = q + _perm(q, p)
                        y = _rsqrt(q)
                        jvec = jnp.full((LANES,), j, jnp.int32)
                        svec = jnp.full((LANES,), sb, jnp.int32)
                        for k in range(4):
                            plsc.store_scatter(
                                sbuf, [svec, rowidx[k], jvec], v[k] * y
                            )

                pltpu.async_copy(
                    sbuf.at[sb, :, pl.ds(0, SUB)],
                    out_hbm.at[pl.ds(u * HIDDEN, HIDDEN), pl.ds(bcol, SUB)],
                    ssems[sb],
                )

                @pl.when(grp < groups - 1)
                def _():
                    pltpu.make_async_copy(
                        idx_hbm.at[pl.ds(irow, 1)], idx_v.at[b], isems[b]
                    ).wait()
                    halve_and_fire(b)

                @pl.when(grp < groups - 2)
                def _():
                    fire_idx(u + 2 * RING, b)
            return carry

        lax.fori_loop(0, groups, group_body, 0)

        for sb in range(2):
            pltpu.make_async_copy(
                sbuf.at[sb, :, pl.ds(0, SUB)],
                out_hbm.at[pl.ds(0, HIDDEN), pl.ds(bcol, SUB)],
                ssems[sb],
            ).wait()

    return gather_norm


def kernel(inputs, table):
    batch, seq = inputs.shape
    idx_t = (
        inputs.T.reshape(seq, batch // SUB, SUB)
        .swapaxes(0, 1)
        .reshape(seq * batch // SUB, SUB)
    )
    table2 = table.reshape(table.shape[0] // 2, 128)
    out = _make_kernel(batch, seq)(idx_t, table2)
    return out.reshape(seq, HIDDEN, batch).transpose(2, 0, 1)
